# Initial kernel scaffold; baseline (speedup 1.0000x reference)
import jax
import jax.numpy as jnp
from jax.experimental import pallas as pl


def kernel(y_pred):
    raise NotImplementedError("write your pallas kernel here")


# speedup vs baseline: 2.5532x; 2.5532x over previous
import functools

import jax
import jax.numpy as jnp
from jax import lax
from jax.experimental import pallas as pl
from jax.experimental.pallas import tpu as pltpu

_N = 16384
_R = 128
_C = 128
_INV = 1.0 / (_N - 1)


def _xor_shift(x, m, axis, low_mask):
    size = x.shape[axis]
    if axis == 0:
        down = jnp.concatenate([x[m:, :], x[:m, :]], axis=0)
        up = jnp.concatenate([x[size - m:, :], x[:size - m, :]], axis=0)
    else:
        down = jnp.concatenate([x[:, m:], x[:, :m]], axis=1)
        up = jnp.concatenate([x[:, size - m:], x[:, :size - m]], axis=1)
    return jnp.where(low_mask, down, up)


def _sort_kernel(x_ref, out_ref):
    yp = jnp.abs(x_ref[...])
    rows = lax.broadcasted_iota(jnp.int32, (_R, _C), 0)
    cols = lax.broadcasted_iota(jnp.int32, (_R, _C), 1)

    key = yp
    val = rows * _C + cols

    kk = 2
    while kk <= _N:
        s = kk // 2
        while s >= 1:
            if s < _C:
                axis, m = 1, s
                low = (cols & s) == 0
            else:
                axis, m = 0, s // _C
                low = (rows & m) == 0
            if kk >= _N:
                keep_min = low
            elif kk < _C:
                keep_min = ((cols & kk) == 0) == low
            else:
                keep_min = ((rows & (kk // _C)) == 0) == low

            pk = _xor_shift(key, m, axis, low)
            pv = _xor_shift(val, m, axis, low)
            t = (pk < key) | ((pk == key) & (pv < val))
            take_p = t == keep_min
            key = jnp.where(take_p, pk, key)
            val = jnp.where(take_p, pv, val)
            s //= 2
        kk *= 2

    terms = jnp.abs(yp - val.astype(jnp.float32) * _INV)
    out_ref[...] = (jnp.sum(terms) * (1.0 / _N)).reshape(1, 1)


@jax.jit
def kernel(y_pred):
    x2d = y_pred.reshape(_R, _C)
    out = pl.pallas_call(
        _sort_kernel,
        out_shape=jax.ShapeDtypeStruct((1, 1), jnp.float32),
    )(x2d)
    return out[0, 0]


# speedup vs baseline: 3.4588x; 1.3547x over previous
import functools

import jax
import jax.numpy as jnp
from jax import lax
from jax.experimental import pallas as pl
from jax.experimental.pallas import tpu as pltpu

_N = 16384
_R = 128
_C = 128
_INV = 1.0 / (_N - 1)


def _xor_shift(x, m, axis, low_mask):
    size = x.shape[axis]
    if axis == 0:
        down = jnp.concatenate([x[m:, :], x[:m, :]], axis=0)
        up = jnp.concatenate([x[size - m:, :], x[:size - m, :]], axis=0)
    else:
        down = jnp.concatenate([x[:, m:], x[:, :m]], axis=1)
        up = jnp.concatenate([x[:, size - m:], x[:, :size - m]], axis=1)
    return jnp.where(low_mask, down, up)


def _sort_kernel(x_ref, out_ref):
    yp = jnp.abs(x_ref[...])
    rows = lax.broadcasted_iota(jnp.int32, (_R, _C), 0)
    cols = lax.broadcasted_iota(jnp.int32, (_R, _C), 1)

    key = yp
    val = rows * _C + cols

    kk = 2
    while kk <= _N:
        s = kk // 2
        while s >= 1:
            if s < _R:
                axis, m = 0, s
                low = (rows & s) == 0
            else:
                axis, m = 1, s // _R
                low = (cols & m) == 0
            if kk >= _N:
                keep_min = low
            elif kk < _R:
                keep_min = ((rows & kk) == 0) == low
            else:
                keep_min = ((cols & (kk // _R)) == 0) == low

            pk = _xor_shift(key, m, axis, low)
            pv = _xor_shift(val, m, axis, low)
            t = (pk < key) | ((pk == key) & (pv < val))
            take_p = t == keep_min
            key = jnp.where(take_p, pk, key)
            val = jnp.where(take_p, pv, val)
            s //= 2
        kk *= 2

    terms = jnp.abs(yp.T - val.astype(jnp.float32) * _INV)
    out_ref[...] = (jnp.sum(terms) * (1.0 / _N)).reshape(1, 1)


@jax.jit
def kernel(y_pred):
    x2d = y_pred.reshape(_R, _C)
    out = pl.pallas_call(
        _sort_kernel,
        out_shape=jax.ShapeDtypeStruct((1, 1), jnp.float32),
    )(x2d)
    return out[0, 0]


# speedup vs baseline: 3.9910x; 1.1539x over previous
import jax
import jax.numpy as jnp
from jax import lax
from jax.experimental import pallas as pl

_N = 16384
_R = 128
_C = 128
_INV = 1.0 / (_N - 1)


def _less(pk, k, pv, v):
    return (pk < k) | ((pk == k) & (pv < v))


def _block_pass(key, val, s):
    nk, nv = [], []
    for base in range(0, _R, 2 * s):
        ak, bk = key[base:base + s], key[base + s:base + 2 * s]
        av, bv = val[base:base + s], val[base + s:base + 2 * s]
        t = _less(bk, ak, bv, av)
        nk.append(jnp.where(t, bk, ak))
        nv.append(jnp.where(t, bv, av))
        nk.append(jnp.where(t, ak, bk))
        nv.append(jnp.where(t, av, bv))
    return jnp.concatenate(nk, axis=0), jnp.concatenate(nv, axis=0)


def _roll_pass(key, val, m, axis, low):
    def xor_shift(x):
        size = x.shape[axis]
        if axis == 0:
            down = jnp.concatenate([x[m:], x[:m]], axis=0)
            up = jnp.concatenate([x[size - m:], x[:size - m]], axis=0)
        else:
            down = jnp.concatenate([x[:, m:], x[:, :m]], axis=1)
            up = jnp.concatenate([x[:, size - m:], x[:, :size - m]], axis=1)
        return jnp.where(low, down, up)

    pk, pv = xor_shift(key), xor_shift(val)
    t = _less(pk, key, pv, val)
    take_p = t == low
    return jnp.where(take_p, pk, key), jnp.where(take_p, pv, val)


def _sort_kernel(x_ref, out_ref):
    yp = jnp.abs(x_ref[...])
    rows = lax.broadcasted_iota(jnp.int32, (_R, _C), 0)
    cols = lax.broadcasted_iota(jnp.int32, (_R, _C), 1)

    key = lax.bitcast_convert_type(yp, jnp.int32)
    val = rows * _C + cols

    low_masks = {}
    for s in (1, 2, 4):
        low_masks[s] = (rows & s) == 0
    for s in (_R, 2 * _R, 4 * _R, 8 * _R, 16 * _R, 32 * _R, 64 * _R):
        low_masks[s] = (cols & (s // _R)) == 0

    def dir_mask(kk):
        if kk >= _N:
            return jnp.zeros((_R, _C), jnp.int32)
        bits = rows & kk if kk < _R else cols & (kk // _R)
        return -(bits != 0).astype(jnp.int32)

    prev_d = jnp.zeros((_R, _C), jnp.int32)
    kk = 2
    while kk <= _N:
        d = dir_mask(kk)
        flip = prev_d ^ d
        key, val, prev_d = key ^ flip, val ^ flip, d
        s = kk // 2
        while s >= 1:
            if s < _R and s >= 8:
                key, val = _block_pass(key, val, s)
            elif s < 8:
                key, val = _roll_pass(key, val, s, 0, low_masks[s])
            else:
                key, val = _roll_pass(key, val, s // _R, 1, low_masks[s])
            s //= 2
        kk *= 2

    terms = jnp.abs(yp.T - val.astype(jnp.float32) * _INV)
    out_ref[...] = (jnp.sum(terms) * (1.0 / _N)).reshape(1, 1)


@jax.jit
def kernel(y_pred):
    x2d = y_pred.reshape(_R, _C)
    out = pl.pallas_call(
        _sort_kernel,
        out_shape=jax.ShapeDtypeStruct((1, 1), jnp.float32),
    )(x2d)
    return out[0, 0]


# speedup vs baseline: 4.0381x; 1.0118x over previous
import numpy as np

import jax
import jax.numpy as jnp
from jax import lax
from jax.experimental import pallas as pl

_N = 16384
_R = 128
_C = 128
_H = 64
_INV = 1.0 / (_N - 1)

def _dir_arr(kk, rows_i, cols_i):
    if kk >= _N:
        return None
    if kk < _R:
        return -((rows_i & kk) != 0).astype(jnp.int32)
    return -((cols_i & (kk // _R)) != 0).astype(jnp.int32)


def _xor_flip(key, val, prev, d):
    f = d if prev is None else (prev if d is None else prev ^ d)
    if f is None:
        return key, val
    return key ^ f, val ^ f


def _less(pk, k, pv, v):
    return (pk < k) | ((pk == k) & (pv < v))


def _block_pass(key, val, s):
    nk, nv = [], []
    for base in range(0, key.shape[0], 2 * s):
        ak, bk = key[base:base + s], key[base + s:base + 2 * s]
        av, bv = val[base:base + s], val[base + s:base + 2 * s]
        t = _less(bk, ak, bv, av)
        nk.append(jnp.where(t, bk, ak))
        nv.append(jnp.where(t, bv, av))
        nk.append(jnp.where(t, ak, bk))
        nv.append(jnp.where(t, av, bv))
    return jnp.concatenate(nk, axis=0), jnp.concatenate(nv, axis=0)


def _roll_pass(key, val, m, axis, low):
    def xor_shift(x):
        size = x.shape[axis]
        if axis == 0:
            down = jnp.concatenate([x[m:], x[:m]], axis=0)
            up = jnp.concatenate([x[size - m:], x[:size - m]], axis=0)
        else:
            down = jnp.concatenate([x[:, m:], x[:, :m]], axis=1)
            up = jnp.concatenate([x[:, size - m:], x[:, :size - m]], axis=1)
        return jnp.where(low, down, up)

    pk, pv = xor_shift(key), xor_shift(val)
    t = _less(pk, key, pv, val)
    take_p = t == low
    return jnp.where(take_p, pk, key), jnp.where(take_p, pv, val)


def _tail(key, val, s, low_sub):
    while s >= 1:
        if s >= 8:
            key, val = _block_pass(key, val, s)
        else:
            key, val = _roll_pass(key, val, s, 0, low_sub[s])
        s //= 2
    return key, val


def _sort_kernel(x_ref, out_ref):
    yp = jnp.abs(x_ref[...])
    rows = lax.broadcasted_iota(jnp.int32, (_R, _C), 0)
    cols = lax.broadcasted_iota(jnp.int32, (_R, _C), 1)
    rows64 = lax.broadcasted_iota(jnp.int32, (_H, _C), 0)
    cols64 = lax.broadcasted_iota(jnp.int32, (_H, _C), 1)

    key = lax.bitcast_convert_type(yp, jnp.int32)
    val = rows * _C + cols

    low_sub = {s: (rows64 & s) == 0 for s in (1, 2, 4)}
    low_lane = {m: (cols & m) == 0 for m in (1, 2, 4, 8, 16, 32, 64)}

    kh = [key[:_H], key[_H:]]
    vh = [val[:_H], val[_H:]]
    for h in (0, 1):
        prev = None
        kk = 2
        while kk <= _H:
            if kk == _H:
                d = None if h == 0 else -jnp.ones((_H, _C), jnp.int32)
            else:
                d = _dir_arr(kk, rows64, cols64)
            kh[h], vh[h] = _xor_flip(kh[h], vh[h], prev, d)
            prev = d
            kh[h], vh[h] = _tail(kh[h], vh[h], kk // 2, low_sub)
            kk *= 2
    key = jnp.concatenate(kh, axis=0)
    val = jnp.concatenate(vh, axis=0)

    prev = _dir_arr(_H, rows, cols)
    kk = _R
    while kk <= _N:
        d = _dir_arr(kk, rows, cols)
        key, val = _xor_flip(key, val, prev, d)
        prev = d
        s = kk // 2
        while s >= _R:
            key, val = _roll_pass(key, val, s // _R, 1, low_lane[s // _R])
            s //= 2
        key, val = _block_pass(key, val, _H)
        kh = [key[:_H], key[_H:]]
        vh = [val[:_H], val[_H:]]
        for h in (0, 1):
            kh[h], vh[h] = _tail(kh[h], vh[h], 32, low_sub)
        key = jnp.concatenate(kh, axis=0)
        val = jnp.concatenate(vh, axis=0)
        kk *= 2

    terms = jnp.abs(yp.T - val.astype(jnp.float32) * _INV)
    out_ref[...] = (jnp.sum(terms) * (1.0 / _N)).reshape(1, 1)


@jax.jit
def kernel(y_pred):
    x2d = y_pred.reshape(_R, _C)
    out = pl.pallas_call(
        _sort_kernel,
        out_shape=jax.ShapeDtypeStruct((1, 1), jnp.float32),
    )(x2d)
    return out[0, 0]


# speedup vs baseline: 5.2736x; 1.3060x over previous
import numpy as np

import jax
import jax.numpy as jnp
from jax import lax
from jax.experimental import pallas as pl

_N = 16384
_R = 128
_C = 128
_H = 64
_INV = 1.0 / (_N - 1)

def _dir_arr(kk, rows_i, cols_i):
    if kk >= _N:
        return None
    if kk < _R:
        return -((rows_i & kk) != 0).astype(jnp.int32)
    return -((cols_i & (kk // _R)) != 0).astype(jnp.int32)


def _xor_flip(key, val, prev, d):
    f = d if prev is None else (prev if d is None else prev ^ d)
    if f is None:
        return key, val
    return key ^ f, val ^ f


def _less(pk, k, pv, v):
    return (pk < k) | ((pk == k) & (pv < v))


def _block_pass(key, val, s):
    nk, nv = [], []
    for base in range(0, key.shape[0], 2 * s):
        ak, bk = key[base:base + s], key[base + s:base + 2 * s]
        av, bv = val[base:base + s], val[base + s:base + 2 * s]
        t = _less(bk, ak, bv, av)
        nk.append(jnp.where(t, bk, ak))
        nv.append(jnp.where(t, bv, av))
        nk.append(jnp.where(t, ak, bk))
        nv.append(jnp.where(t, av, bv))
    return jnp.concatenate(nk, axis=0), jnp.concatenate(nv, axis=0)


def _roll_pass(key, val, m, axis, low):
    if axis == 0 and m == 4:
        def butterfly(x):
            pieces = []
            for base in range(0, x.shape[0], 2 * m):
                pieces.append(x[base + m:base + 2 * m])
                pieces.append(x[base:base + m])
            return jnp.concatenate(pieces, axis=0)
        pk, pv = butterfly(key), butterfly(val)
    elif axis == 0:
        perm8 = lax.broadcasted_iota(jnp.int32, (8, key.shape[1]), 0) ^ m
        def sub_butterfly(x):
            return jnp.concatenate(
                [jnp.take_along_axis(x[b:b + 8], perm8, axis=0)
                 for b in range(0, x.shape[0], 8)], axis=0)
        pk, pv = sub_butterfly(key), sub_butterfly(val)
    else:
        cols_i = lax.broadcasted_iota(jnp.int32, key.shape, 1)
        perm = cols_i ^ m
        pk = jnp.take_along_axis(key, perm, axis=1)
        pv = jnp.take_along_axis(val, perm, axis=1)

    t = _less(pk, key, pv, val)
    take_p = t == low
    return jnp.where(take_p, pk, key), jnp.where(take_p, pv, val)


def _tail(key, val, s, low_sub):
    while s >= 1:
        if s >= 8:
            key, val = _block_pass(key, val, s)
        else:
            key, val = _roll_pass(key, val, s, 0, low_sub[s])
        s //= 2
    return key, val


def _sort_kernel(x_ref, out_ref):
    yp = jnp.abs(x_ref[...])
    rows = lax.broadcasted_iota(jnp.int32, (_R, _C), 0)
    cols = lax.broadcasted_iota(jnp.int32, (_R, _C), 1)
    rows64 = lax.broadcasted_iota(jnp.int32, (_H, _C), 0)
    cols64 = lax.broadcasted_iota(jnp.int32, (_H, _C), 1)

    key = lax.bitcast_convert_type(yp, jnp.int32)
    val = rows * _C + cols

    low_sub = {s: (rows64 & s) == 0 for s in (1, 2, 4)}
    low_lane = {m: (cols & m) == 0 for m in (1, 2, 4, 8, 16, 32, 64)}

    kh = [key[:_H], key[_H:]]
    vh = [val[:_H], val[_H:]]
    for h in (0, 1):
        prev = None
        kk = 2
        while kk <= _H:
            if kk == _H:
                d = None if h == 0 else -jnp.ones((_H, _C), jnp.int32)
            else:
                d = _dir_arr(kk, rows64, cols64)
            kh[h], vh[h] = _xor_flip(kh[h], vh[h], prev, d)
            prev = d
            kh[h], vh[h] = _tail(kh[h], vh[h], kk // 2, low_sub)
            kk *= 2
    key = jnp.concatenate(kh, axis=0)
    val = jnp.concatenate(vh, axis=0)

    prev = _dir_arr(_H, rows, cols)
    kk = _R
    while kk <= _N:
        d = _dir_arr(kk, rows, cols)
        key, val = _xor_flip(key, val, prev, d)
        prev = d
        s = kk // 2
        while s >= _R:
            key, val = _roll_pass(key, val, s // _R, 1, low_lane[s // _R])
            s //= 2
        key, val = _block_pass(key, val, _H)
        kh = [key[:_H], key[_H:]]
        vh = [val[:_H], val[_H:]]
        for h in (0, 1):
            kh[h], vh[h] = _tail(kh[h], vh[h], 32, low_sub)
        key = jnp.concatenate(kh, axis=0)
        val = jnp.concatenate(vh, axis=0)
        kk *= 2

    terms = jnp.abs(yp.T - val.astype(jnp.float32) * _INV)
    out_ref[...] = (jnp.sum(terms) * (1.0 / _N)).reshape(1, 1)


@jax.jit
def kernel(y_pred):
    x2d = y_pred.reshape(_R, _C)
    out = pl.pallas_call(
        _sort_kernel,
        out_shape=jax.ShapeDtypeStruct((1, 1), jnp.float32),
    )(x2d)
    return out[0, 0]
